# Initial kernel scaffold; baseline (speedup 1.0000x reference)
import jax
import jax.numpy as jnp
from jax.experimental import pallas as pl


def kernel(inputs, edge_index, W1, b1, W2, b2, Wf, bf, fw):
    raise NotImplementedError("write your pallas kernel here")


# speedup vs baseline: 1.6941x; 1.6941x over previous
import functools

import jax
import jax.numpy as jnp
from jax import lax
from jax.experimental import pallas as pl
from jax.experimental.pallas import tpu as pltpu
from jax.experimental.pallas import tpu_sc as plsc

NC = 2
NS = 16
NW = NC * NS
CHUNK = 128
LANES = 16
BLK = 1024


def _round_up(v, m):
    return -(-v // m) * m


def _flat_fill(ref, nrows, ncols, val):
    per_row = ncols // LANES
    vec = jnp.full((LANES,), val, jnp.float32)

    def body(k, _):
        i = k // per_row
        j = (k % per_row) * LANES
        ref[i, pl.ds(j, LANES)] = vec
        return 0

    lax.fori_loop(0, nrows * per_row, body, 0, unroll=False)


def _geom(n_nodes):
    half_n = _round_up(n_nodes, 2 * BLK) // 2
    rows_sub = _round_up(-(-(half_n + 1) // NS), 64)
    n_acc = NS * rows_sub
    return half_n, rows_sub, n_acc


def _remap_chunk(dst_v, base, half_n):
    for j in range(CHUNK // LANES):
        v = dst_v[pl.ds(j * LANES, LANES)] - base
        ok = (v >= 0) & (v < half_n)
        dst_v[pl.ds(j * LANES, LANES)] = jnp.where(ok, v, half_n)


def _make_count_kernel(n_nodes, n_chunks):
    half_n, rows_sub, n_acc = _geom(n_nodes)
    zb = 64
    cps = n_chunks // NS
    e_pad = n_chunks * CHUNK
    mesh = plsc.VectorSubcoreMesh(core_axis_name="c", subcore_axis_name="s")

    @functools.partial(
        pl.kernel,
        out_type=jax.ShapeDtypeStruct((2, NC, n_acc, 128), jnp.float32),
        mesh=mesh,
        scratch_types=[
            pltpu.VMEM((CHUNK,), jnp.int32),
            pltpu.VMEM((CHUNK, 128), jnp.float32),
            pltpu.VMEM((zb, 128), jnp.float32),
            pltpu.VMEM_SHARED((n_acc, 128), jnp.float32),
        ],
    )
    def count_kernel(idx_hbm, out_hbm, idx_v, ones_v, zero_v, acc):
        cid = lax.axis_index("c")
        sid = lax.axis_index("s")
        _flat_fill(ones_v, CHUNK, 128, 1.0)
        _flat_fill(zero_v, zb, 128, 0.0)
        base = cid * half_n

        for ph in range(2):
            for k in range(rows_sub // zb):
                pltpu.sync_copy(zero_v,
                                acc.at[pl.ds(sid * rows_sub + k * zb, zb)])
            plsc.subcore_barrier()
            base0 = ph * e_pad + sid * cps * CHUNK

            def body(k, _):
                pltpu.sync_copy(idx_hbm.at[pl.ds(base0 + k * CHUNK, CHUNK)],
                                idx_v)
                _remap_chunk(idx_v, base, half_n)
                pltpu.sync_copy(ones_v, acc.at[idx_v], add=True)
                return 0

            lax.fori_loop(0, cps, body, 0, unroll=False)
            plsc.subcore_barrier()
            pltpu.sync_copy(acc.at[pl.ds(sid * rows_sub, rows_sub)],
                            out_hbm.at[ph, cid, pl.ds(sid * rows_sub,
                                                      rows_sub)])

    return count_kernel


def _make_agg_kernel(n_nodes, h_dim, n_chunks):
    half_n, rows_sub, n_acc = _geom(n_nodes)
    zb = 64
    cps = n_chunks // NS
    mesh = plsc.VectorSubcoreMesh(core_axis_name="c", subcore_axis_name="s")

    @functools.partial(
        pl.kernel,
        out_type=jax.ShapeDtypeStruct((NC, n_acc, h_dim), jnp.float32),
        mesh=mesh,
        scratch_types=[
            pltpu.VMEM((CHUNK,), jnp.int32),
            pltpu.VMEM((CHUNK,), jnp.int32),
            pltpu.VMEM((CHUNK, h_dim), jnp.float32),
            pltpu.VMEM((zb, h_dim), jnp.float32),
            pltpu.VMEM_SHARED((n_acc, h_dim), jnp.float32),
            pltpu.SemaphoreType.DMA,
        ],
    )
    def agg_kernel(h_hbm, src_hbm, dst_hbm, out_hbm,
                   src_v, dst_v, rows_v, zero_v, acc, sem):
        cid = lax.axis_index("c")
        sid = lax.axis_index("s")

        _flat_fill(zero_v, zb, h_dim, 0.0)
        for k in range(rows_sub // zb):
            pltpu.sync_copy(zero_v, acc.at[pl.ds(sid * rows_sub + k * zb, zb)])
        plsc.subcore_barrier()
        base0 = sid * cps * CHUNK
        base = cid * half_n

        def body(k, _):
            pltpu.sync_copy(src_hbm.at[pl.ds(base0 + k * CHUNK, CHUNK)], src_v)
            pltpu.sync_copy(dst_hbm.at[pl.ds(base0 + k * CHUNK, CHUNK)], dst_v)
            gather = pltpu.async_copy(h_hbm.at[src_v], rows_v, sem)
            _remap_chunk(dst_v, base, half_n)
            gather.wait()
            pltpu.sync_copy(rows_v, acc.at[dst_v], add=True)
            return 0

        lax.fori_loop(0, cps, body, 0, unroll=False)
        plsc.subcore_barrier()
        pltpu.sync_copy(acc.at[pl.ds(sid * rows_sub, rows_sub)],
                        out_hbm.at[cid, pl.ds(sid * rows_sub, rows_sub)])

    return agg_kernel


def _inv_sqrt_deg(dref):
    d = dref[0, 0, :, 0:1]
    return jnp.where(d > 0, lax.rsqrt(jnp.maximum(d, 1.0)), 0.0)


def _tc1_body(x_ref, w_ref, ds_ref, out_ref):
    inv_s = _inv_sqrt_deg(ds_ref)
    out_ref[...] = jnp.dot(x_ref[...] * inv_s, w_ref[...],
                           preferred_element_type=jnp.float32)


def _tc2_body(p_ref, dd_ref, ds_ref, b_ref, w_ref, out_ref):
    inv_d = _inv_sqrt_deg(dd_ref)
    inv_s = _inv_sqrt_deg(ds_ref)
    h = jnp.maximum(p_ref[0] * inv_d + b_ref[...], 0.0)
    out_ref[...] = jnp.dot(h * inv_s, w_ref[...],
                           preferred_element_type=jnp.float32)


def _tc3_body(p_ref, dd_ref, b_ref, wf_ref, bf_ref, fw_ref, out_ref):
    inv_d = _inv_sqrt_deg(dd_ref)
    h = jnp.maximum(p_ref[0] * inv_d + b_ref[...], 0.0)
    fwv = fw_ref[...]
    m = jnp.max(fwv, axis=0, keepdims=True)
    e = jnp.exp(fwv - m)
    w = e / jnp.sum(e, axis=0, keepdims=True)
    wc = jnp.sum(wf_ref[...] * w[:, :, None], axis=0)
    bc = jnp.sum(bf_ref[...] * w, axis=0, keepdims=True)
    logits = jnp.dot(h, wc, preferred_element_type=jnp.float32) + bc
    mx = jnp.max(logits, axis=-1, keepdims=True)
    lse = mx + jnp.log(jnp.sum(jnp.exp(logits - mx), axis=-1, keepdims=True))
    out_ref[...] = logits - lse


def kernel(inputs, edge_index, W1, b1, W2, b2, Wf, bf, fw):
    n, d_in = inputs.shape
    e = edge_index.shape[1]
    h_dim = W1.shape[1]
    nel, _, c_dim = Wf.shape
    half_n, _, _ = _geom(n)

    cpt = -(-e // (NW * CHUNK))
    cpt = ((cpt + 7) // 8) * 8
    e_pad = NW * CHUNK * cpt
    pad = e_pad - e

    src = edge_index[0]
    dst = edge_index[1]
    n_chunks = e_pad // CHUNK
    dst_p = jnp.concatenate([dst, jnp.full((pad,), n, jnp.int32)])
    src_deg = jnp.concatenate([src, jnp.full((pad,), n, jnp.int32)])
    src_agg = jnp.concatenate([src, jnp.zeros((pad,), jnp.int32)])
    idx_all = jnp.concatenate([src_deg, dst_p])

    count_kernel = _make_count_kernel(n, n_chunks)
    deg = count_kernel(idx_all)
    agg_kernel = _make_agg_kernel(n, h_dim, n_chunks)

    grid = (-(-n // BLK),)
    hb = half_n // BLK
    feat_shape = jax.ShapeDtypeStruct((n, h_dim), jnp.float32)
    feat_spec = pl.BlockSpec((BLK, h_dim), lambda i: (i, 0))
    part_spec = pl.BlockSpec((1, BLK, h_dim), lambda i: (i // hb, i % hb, 0))
    degs_spec = pl.BlockSpec((1, 1, BLK, 128),
                             lambda i: (0, i // hb, i % hb, 0))
    degd_spec = pl.BlockSpec((1, 1, BLK, 128),
                             lambda i: (1, i // hb, i % hb, 0))

    xw = pl.pallas_call(
        _tc1_body,
        grid=grid,
        in_specs=[
            pl.BlockSpec((BLK, d_in), lambda i: (i, 0)),
            pl.BlockSpec((d_in, h_dim), lambda i: (0, 0)),
            degs_spec,
        ],
        out_specs=feat_spec,
        out_shape=feat_shape,
    )(inputs, W1, deg)

    a1 = agg_kernel(xw, src_agg, dst_p)

    hw = pl.pallas_call(
        _tc2_body,
        grid=grid,
        in_specs=[
            part_spec,
            degd_spec,
            degs_spec,
            pl.BlockSpec((1, h_dim), lambda i: (0, 0)),
            pl.BlockSpec((h_dim, h_dim), lambda i: (0, 0)),
        ],
        out_specs=feat_spec,
        out_shape=feat_shape,
    )(a1, deg, deg, b1.reshape(1, h_dim), W2)

    a2 = agg_kernel(hw, src_agg, dst_p)

    out = pl.pallas_call(
        _tc3_body,
        grid=grid,
        in_specs=[
            part_spec,
            degd_spec,
            pl.BlockSpec((1, h_dim), lambda i: (0, 0)),
            pl.BlockSpec((nel, h_dim, c_dim), lambda i: (0, 0, 0)),
            pl.BlockSpec((nel, c_dim), lambda i: (0, 0)),
            pl.BlockSpec((nel, 1), lambda i: (0, 0)),
        ],
        out_specs=pl.BlockSpec((BLK, c_dim), lambda i: (i, 0)),
        out_shape=jax.ShapeDtypeStruct((n, c_dim), jnp.float32),
    )(a2, deg, b2.reshape(1, h_dim), Wf, bf, fw.reshape(nel, 1))

    return out


# speedup vs baseline: 1.8905x; 1.1159x over previous
import functools

import jax
import jax.numpy as jnp
from jax import lax
from jax.experimental import pallas as pl
from jax.experimental.pallas import tpu as pltpu
from jax.experimental.pallas import tpu_sc as plsc

NC = 2
NS = 16
NW = NC * NS
CHUNK = 128
LANES = 16
BLK = 1024
NBUF = 2


def _round_up(v, m):
    return -(-v // m) * m


def _flat_fill(ref, nrows, ncols, val):
    per_row = ncols // LANES
    vec = jnp.full((LANES,), val, jnp.float32)

    def body(k, _):
        i = k // per_row
        j = (k % per_row) * LANES
        ref[i, pl.ds(j, LANES)] = vec
        return 0

    lax.fori_loop(0, nrows * per_row, body, 0, unroll=False)


def _geom(n_nodes):
    half_n = _round_up(n_nodes, 2 * BLK) // 2
    rows_sub = _round_up(-(-(half_n + 1) // NS), 64)
    n_acc = NS * rows_sub
    return half_n, rows_sub, n_acc


def _remap_all(dst_v, count, base, half_n):
    def body(j, _):
        v = dst_v[pl.ds(j * LANES, LANES)] - base
        ok = (v >= 0) & (v < half_n)
        dst_v[pl.ds(j * LANES, LANES)] = jnp.where(ok, v, half_n)
        return 0

    lax.fori_loop(0, count // LANES, body, 0, unroll=False)


def _make_count_kernel(n_nodes, n_chunks):
    half_n, rows_sub, n_acc = _geom(n_nodes)
    zb = 64
    cps = n_chunks // NS
    e_pad = n_chunks * CHUNK
    mesh = plsc.VectorSubcoreMesh(core_axis_name="c", subcore_axis_name="s")

    @functools.partial(
        pl.kernel,
        out_type=jax.ShapeDtypeStruct((2, NC, n_acc, 128), jnp.float32),
        mesh=mesh,
        scratch_types=[
            pltpu.VMEM((cps * CHUNK,), jnp.int32),
            pltpu.VMEM((CHUNK, 128), jnp.float32),
            pltpu.VMEM((zb, 128), jnp.float32),
            pltpu.VMEM_SHARED((n_acc, 128), jnp.float32),
            pltpu.SemaphoreType.DMA,
        ],
    )
    def count_kernel(idx_hbm, out_hbm, idx_v, ones_v, zero_v, acc, sem):
        cid = lax.axis_index("c")
        sid = lax.axis_index("s")
        _flat_fill(ones_v, CHUNK, 128, 1.0)
        _flat_fill(zero_v, zb, 128, 0.0)
        base = cid * half_n
        nfire = 8

        for ph in range(2):
            for k in range(rows_sub // zb):
                pltpu.sync_copy(zero_v,
                                acc.at[pl.ds(sid * rows_sub + k * zb, zb)])
            base0 = ph * e_pad + sid * cps * CHUNK
            pltpu.sync_copy(idx_hbm.at[pl.ds(base0, cps * CHUNK)], idx_v)
            _remap_all(idx_v, cps * CHUNK, base, half_n)
            plsc.subcore_barrier()

            def group(g, _):
                descr = []
                for b in range(nfire):
                    k = g * nfire + b
                    descr.append(pltpu.async_copy(
                        ones_v, acc.at[idx_v.at[pl.ds(k * CHUNK, CHUNK)]],
                        sem, add=True))
                for d in descr:
                    d.wait()
                return 0

            lax.fori_loop(0, cps // nfire, group, 0, unroll=False)
            plsc.subcore_barrier()
            pltpu.sync_copy(acc.at[pl.ds(sid * rows_sub, rows_sub)],
                            out_hbm.at[ph, cid, pl.ds(sid * rows_sub,
                                                      rows_sub)])
            plsc.subcore_barrier()

    return count_kernel


def _make_agg_kernel(n_nodes, h_dim, n_chunks):
    half_n, rows_sub, n_acc = _geom(n_nodes)
    zb = 64
    cps = n_chunks // NS
    mesh = plsc.VectorSubcoreMesh(core_axis_name="c", subcore_axis_name="s")

    @functools.partial(
        pl.kernel,
        out_type=jax.ShapeDtypeStruct((NC, n_acc, h_dim), jnp.float32),
        mesh=mesh,
        scratch_types=[
            pltpu.VMEM((cps * CHUNK,), jnp.int32),
            pltpu.VMEM((cps * CHUNK,), jnp.int32),
            [pltpu.VMEM((CHUNK, h_dim), jnp.float32) for _ in range(NBUF)],
            pltpu.VMEM((zb, h_dim), jnp.float32),
            pltpu.VMEM_SHARED((n_acc, h_dim), jnp.float32),
            [pltpu.SemaphoreType.DMA for _ in range(NBUF)],
        ],
    )
    def agg_kernel(h_hbm, src_hbm, dst_hbm, out_hbm,
                   src_v, dst_v, rows_v, zero_v, acc, sems):
        cid = lax.axis_index("c")
        sid = lax.axis_index("s")

        _flat_fill(zero_v, zb, h_dim, 0.0)
        for k in range(rows_sub // zb):
            pltpu.sync_copy(zero_v, acc.at[pl.ds(sid * rows_sub + k * zb, zb)])
        base0 = sid * cps * CHUNK
        base = cid * half_n
        pltpu.sync_copy(src_hbm.at[pl.ds(base0, cps * CHUNK)], src_v)
        pltpu.sync_copy(dst_hbm.at[pl.ds(base0, cps * CHUNK)], dst_v)
        _remap_all(dst_v, cps * CHUNK, base, half_n)
        plsc.subcore_barrier()

        def gather(k, b):
            pltpu.async_copy(
                h_hbm.at[src_v.at[pl.ds(k * CHUNK, CHUNK)]], rows_v[b],
                sems[b])

        def gather_wait(b):
            pltpu.make_async_copy(
                h_hbm.at[src_v.at[pl.ds(0, CHUNK)]], rows_v[b],
                sems[b]).wait()

        for b in range(NBUF):
            gather(b, b)

        ngroups = cps // NBUF

        def group(g, _):
            for b in range(NBUF):
                k = g * NBUF + b
                gather_wait(b)
                pltpu.sync_copy(rows_v[b],
                                acc.at[dst_v.at[pl.ds(k * CHUNK, CHUNK)]],
                                add=True)

                @pl.when(g + 1 < ngroups)
                def _():
                    gather(k + NBUF, b)
            return 0

        lax.fori_loop(0, ngroups, group, 0, unroll=False)
        plsc.subcore_barrier()
        pltpu.sync_copy(acc.at[pl.ds(sid * rows_sub, rows_sub)],
                        out_hbm.at[cid, pl.ds(sid * rows_sub, rows_sub)])

    return agg_kernel


def _inv_sqrt_deg(dref):
    d = dref[0, 0, :, 0:1]
    return jnp.where(d > 0, lax.rsqrt(jnp.maximum(d, 1.0)), 0.0)


def _tc1_body(x_ref, w_ref, ds_ref, out_ref):
    inv_s = _inv_sqrt_deg(ds_ref)
    out_ref[...] = jnp.dot(x_ref[...] * inv_s, w_ref[...],
                           preferred_element_type=jnp.float32)


def _tc2_body(p_ref, dd_ref, ds_ref, b_ref, w_ref, out_ref):
    inv_d = _inv_sqrt_deg(dd_ref)
    inv_s = _inv_sqrt_deg(ds_ref)
    h = jnp.maximum(p_ref[0] * inv_d + b_ref[...], 0.0)
    out_ref[...] = jnp.dot(h * inv_s, w_ref[...],
                           preferred_element_type=jnp.float32)


def _tc3_body(p_ref, dd_ref, b_ref, wf_ref, bf_ref, fw_ref, out_ref):
    inv_d = _inv_sqrt_deg(dd_ref)
    h = jnp.maximum(p_ref[0] * inv_d + b_ref[...], 0.0)
    fwv = fw_ref[...]
    m = jnp.max(fwv, axis=0, keepdims=True)
    e = jnp.exp(fwv - m)
    w = e / jnp.sum(e, axis=0, keepdims=True)
    wc = jnp.sum(wf_ref[...] * w[:, :, None], axis=0)
    bc = jnp.sum(bf_ref[...] * w, axis=0, keepdims=True)
    logits = jnp.dot(h, wc, preferred_element_type=jnp.float32) + bc
    mx = jnp.max(logits, axis=-1, keepdims=True)
    lse = mx + jnp.log(jnp.sum(jnp.exp(logits - mx), axis=-1, keepdims=True))
    out_ref[...] = logits - lse


def kernel(inputs, edge_index, W1, b1, W2, b2, Wf, bf, fw):
    n, d_in = inputs.shape
    e = edge_index.shape[1]
    h_dim = W1.shape[1]
    nel, _, c_dim = Wf.shape
    half_n, _, _ = _geom(n)

    cpt = -(-e // (NW * CHUNK))
    cpt = ((cpt + 7) // 8) * 8
    e_pad = NW * CHUNK * cpt
    pad = e_pad - e

    src = edge_index[0]
    dst = edge_index[1]
    n_chunks = e_pad // CHUNK
    dst_p = jnp.concatenate([dst, jnp.full((pad,), n, jnp.int32)])
    src_deg = jnp.concatenate([src, jnp.full((pad,), n, jnp.int32)])
    src_agg = jnp.concatenate([src, jnp.zeros((pad,), jnp.int32)])
    idx_all = jnp.concatenate([src_deg, dst_p])

    count_kernel = _make_count_kernel(n, n_chunks)
    deg = count_kernel(idx_all)
    agg_kernel = _make_agg_kernel(n, h_dim, n_chunks)

    grid = (-(-n // BLK),)
    hb = half_n // BLK
    feat_shape = jax.ShapeDtypeStruct((n, h_dim), jnp.float32)
    feat_spec = pl.BlockSpec((BLK, h_dim), lambda i: (i, 0))
    part_spec = pl.BlockSpec((1, BLK, h_dim), lambda i: (i // hb, i % hb, 0))
    degs_spec = pl.BlockSpec((1, 1, BLK, 128),
                             lambda i: (0, i // hb, i % hb, 0))
    degd_spec = pl.BlockSpec((1, 1, BLK, 128),
                             lambda i: (1, i // hb, i % hb, 0))

    xw = pl.pallas_call(
        _tc1_body,
        grid=grid,
        in_specs=[
            pl.BlockSpec((BLK, d_in), lambda i: (i, 0)),
            pl.BlockSpec((d_in, h_dim), lambda i: (0, 0)),
            degs_spec,
        ],
        out_specs=feat_spec,
        out_shape=feat_shape,
    )(inputs, W1, deg)

    a1 = agg_kernel(xw, src_agg, dst_p)

    hw = pl.pallas_call(
        _tc2_body,
        grid=grid,
        in_specs=[
            part_spec,
            degd_spec,
            degs_spec,
            pl.BlockSpec((1, h_dim), lambda i: (0, 0)),
            pl.BlockSpec((h_dim, h_dim), lambda i: (0, 0)),
        ],
        out_specs=feat_spec,
        out_shape=feat_shape,
    )(a1, deg, deg, b1.reshape(1, h_dim), W2)

    a2 = agg_kernel(hw, src_agg, dst_p)

    out = pl.pallas_call(
        _tc3_body,
        grid=grid,
        in_specs=[
            part_spec,
            degd_spec,
            pl.BlockSpec((1, h_dim), lambda i: (0, 0)),
            pl.BlockSpec((nel, h_dim, c_dim), lambda i: (0, 0, 0)),
            pl.BlockSpec((nel, c_dim), lambda i: (0, 0)),
            pl.BlockSpec((nel, 1), lambda i: (0, 0)),
        ],
        out_specs=pl.BlockSpec((BLK, c_dim), lambda i: (i, 0)),
        out_shape=jax.ShapeDtypeStruct((n, c_dim), jnp.float32),
    )(a2, deg, b2.reshape(1, h_dim), Wf, bf, fw.reshape(nel, 1))

    return out
